# baseline (speedup 1.0000x reference)
import functools

import jax
import jax.numpy as jnp
from jax import lax
from jax.experimental import pallas as pl
from jax.experimental.pallas import tpu as pltpu
from jax.experimental.pallas import tpu_sc as plsc

VOCAB = 100000
D = 64
N_HYP = 4
BATCH = 1024
SEQ = 200

C = N_HYP * D
T = BATCH * SEQ
NW = 32
CHUNK = 128
NCHUNKS = T // CHUNK
NCH = NCHUNKS // NW


def _make_sc_kernel():
    mesh = plsc.VectorSubcoreMesh(core_axis_name="c", subcore_axis_name="s")

    @functools.partial(
        pl.kernel,
        mesh=mesh,
        out_type=jax.ShapeDtypeStruct((T, C), jnp.float32),
        scratch_types=[
            pltpu.VMEM((CHUNK,), jnp.int32),
            pltpu.VMEM((CHUNK, C), jnp.float32),
            pltpu.VMEM((C,), jnp.float32),
            pltpu.SemaphoreType.DMA,
        ],
    )
    def k(tbl_hbm, idx_hbm, coef_hbm, out_hbm, idx_v, rows_v, coef_v, sem):
        nc = 2
        wid = lax.axis_index("s") * nc + lax.axis_index("c")

        pltpu.sync_copy(coef_hbm, coef_v)
        coef_reg = [coef_v[pl.ds(16 * h, 16)] for h in range(16)]

        def chunk_body(c, carry):
            gc = wid * NCH + c
            pltpu.sync_copy(idx_hbm.at[gc], idx_v)
            pltpu.async_copy(tbl_hbm.at[idx_v], rows_v, sem).wait()

            def mul_body(t, _):
                for h in range(16):
                    sl = pl.ds(16 * h, 16)
                    rows_v[t, sl] = rows_v[t, sl] * coef_reg[h]
                return _
            lax.fori_loop(0, CHUNK, mul_body, 0)

            pltpu.sync_copy(rows_v, out_hbm.at[pl.ds(gc * CHUNK, CHUNK)])
            return carry

        lax.fori_loop(0, NCH, chunk_body, 0)

    return k


_sc_call = _make_sc_kernel()


_SBLK = 8
_BBLK = 256


def _tc_transpose_body(g_ref, out_ref):
    out_ref[...] = jnp.transpose(g_ref[...], (1, 2, 0))


_tc_transpose = pl.pallas_call(
    _tc_transpose_body,
    grid=(SEQ // _SBLK, BATCH // _BBLK),
    in_specs=[pl.BlockSpec((_BBLK, _SBLK, C), lambda j, kb: (kb, j, 0))],
    out_specs=pl.BlockSpec((_SBLK, C, _BBLK), lambda j, kb: (j, 0, kb)),
    out_shape=jax.ShapeDtypeStruct((SEQ, C, BATCH), jnp.float32),
)


def kernel(x, tables, phases, amplitudes):
    xf = x.reshape(NCHUNKS, CHUNK).astype(jnp.int32)
    tbl = tables.transpose(1, 0, 2).reshape(VOCAB, C)
    coef = (jnp.cos(phases) * amplitudes[:, None]).astype(jnp.float32)
    coef = coef.reshape(C)
    g = _sc_call(tbl, xf, coef)
    out = _tc_transpose(g.reshape(BATCH, SEQ, C))
    out = out.reshape(SEQ, N_HYP, D, BATCH)
    return out.transpose(3, 0, 1, 2)


# speedup vs baseline: 3.7877x; 3.7877x over previous
import functools

import jax
import jax.numpy as jnp
from jax import lax
from jax.experimental import pallas as pl
from jax.experimental.pallas import tpu as pltpu
from jax.experimental.pallas import tpu_sc as plsc

VOCAB = 100000
D = 64
N_HYP = 4
BATCH = 1024
SEQ = 200

C = N_HYP * D
T = BATCH * SEQ
NW = 32
CHUNK = 128
NCHUNKS = T // CHUNK
NCH = NCHUNKS // NW


def _make_sc_kernel():
    mesh = plsc.VectorSubcoreMesh(core_axis_name="c", subcore_axis_name="s")

    @functools.partial(
        pl.kernel,
        mesh=mesh,
        out_type=jax.ShapeDtypeStruct((T, C), jnp.float32),
        scratch_types=[
            pltpu.VMEM((CHUNK,), jnp.int32),
            pltpu.VMEM((CHUNK, C), jnp.float32),
            pltpu.VMEM((C,), jnp.float32),
            pltpu.SemaphoreType.DMA,
        ],
    )
    def k(tbl_hbm, xt_hbm, coef_hbm, out_hbm, idx_v, rows_v, coef_v, sem):
        nc = 2
        wid = lax.axis_index("s") * nc + lax.axis_index("c")

        pltpu.sync_copy(coef_hbm, coef_v)
        coef_reg = [coef_v[pl.ds(16 * h, 16)] for h in range(16)]

        def chunk_body(c, carry):
            g = wid * NCH + c
            s = g // (BATCH // CHUNK)
            b0 = (g % (BATCH // CHUNK)) * CHUNK
            pltpu.sync_copy(xt_hbm.at[s, pl.ds(b0, CHUNK)], idx_v)
            pltpu.async_copy(tbl_hbm.at[idx_v], rows_v, sem).wait()

            def mul_body(t, _):
                for h in range(16):
                    sl = pl.ds(16 * h, 16)
                    rows_v[t, sl] = rows_v[t, sl] * coef_reg[h]
                return _
            lax.fori_loop(0, CHUNK, mul_body, 0)

            pltpu.sync_copy(rows_v, out_hbm.at[pl.ds(s * BATCH + b0, CHUNK)])
            return carry

        lax.fori_loop(0, NCH, chunk_body, 0)

    return k


_sc_call = _make_sc_kernel()


def _tc_transpose_body(g_ref, out_ref):
    out_ref[0] = g_ref[0].T


_tc_transpose = pl.pallas_call(
    _tc_transpose_body,
    grid=(SEQ,),
    in_specs=[pl.BlockSpec((1, BATCH, C), lambda j: (j, 0, 0))],
    out_specs=pl.BlockSpec((1, C, BATCH), lambda j: (j, 0, 0)),
    out_shape=jax.ShapeDtypeStruct((SEQ, C, BATCH), jnp.float32),
)


def kernel(x, tables, phases, amplitudes):
    xt = x.T.astype(jnp.int32)
    tbl = tables.transpose(1, 0, 2).reshape(VOCAB, C)
    coef = (jnp.cos(phases) * amplitudes[:, None]).astype(jnp.float32)
    coef = coef.reshape(C)
    g = _sc_call(tbl, xt, coef)
    out = _tc_transpose(g.reshape(SEQ, BATCH, C))
    out = out.reshape(SEQ, N_HYP, D, BATCH)
    return out.transpose(3, 0, 1, 2)


# speedup vs baseline: 4.6862x; 1.2372x over previous
import functools

import jax
import jax.numpy as jnp
from jax import lax
from jax.experimental import pallas as pl
from jax.experimental.pallas import tpu as pltpu
from jax.experimental.pallas import tpu_sc as plsc

VOCAB = 100000
D = 64
N_HYP = 4
BATCH = 1024
SEQ = 200

C = N_HYP * D
NW = 32
CHUNK = 128
NBBLK = BATCH // CHUNK
NSLICE = 5
SSEQ = SEQ // NSLICE
UPW = SSEQ * NBBLK // NW


def _make_sc_kernel(s0):
    mesh = plsc.VectorSubcoreMesh(core_axis_name="c", subcore_axis_name="s")

    @functools.partial(
        pl.kernel,
        mesh=mesh,
        out_type=jax.ShapeDtypeStruct((SSEQ * BATCH, C), jnp.float32),
        scratch_types=[
            pltpu.VMEM((CHUNK,), jnp.int32),
            pltpu.VMEM((CHUNK, C), jnp.float32),
            pltpu.VMEM((C,), jnp.float32),
            pltpu.SemaphoreType.DMA,
        ],
    )
    def k(tbl_hbm, xt_hbm, coef_hbm, out_hbm, idx_v, rows_v, coef_v, sem):
        nc = 2
        wid = lax.axis_index("s") * nc + lax.axis_index("c")

        pltpu.sync_copy(coef_hbm, coef_v)
        coef_reg = [coef_v[pl.ds(16 * h, 16)] for h in range(16)]

        def chunk_body(c, carry):
            g = wid * UPW + c
            s = g // NBBLK
            b0 = (g % NBBLK) * CHUNK
            pltpu.sync_copy(xt_hbm.at[s0 + s, pl.ds(b0, CHUNK)], idx_v)
            pltpu.async_copy(tbl_hbm.at[idx_v], rows_v, sem).wait()

            def mul_body(t, _):
                for h in range(16):
                    sl = pl.ds(16 * h, 16)
                    rows_v[t, sl] = rows_v[t, sl] * coef_reg[h]
                return _
            lax.fori_loop(0, CHUNK, mul_body, 0)

            pltpu.sync_copy(rows_v, out_hbm.at[pl.ds(s * BATCH + b0, CHUNK)])
            return carry

        lax.fori_loop(0, UPW, chunk_body, 0)

    return k


_sc_call = [_make_sc_kernel(i * SSEQ) for i in range(NSLICE)]


def _tc_transpose_body_first(g_ref, out_ref):
    out_ref[0] = g_ref[0].T


def _tc_transpose_body(carry_ref, g_ref, out_ref):
    del carry_ref
    out_ref[0] = g_ref[0].T


def _make_tc_transpose(s0, first):
    g_spec = pl.BlockSpec((1, BATCH, C), lambda j: (j, 0, 0))
    out_spec = pl.BlockSpec((1, C, BATCH), lambda j: (s0 + j, 0, 0))
    out_shape = jax.ShapeDtypeStruct((SEQ, C, BATCH), jnp.float32)
    if first:
        return pl.pallas_call(
            _tc_transpose_body_first,
            grid=(SSEQ,),
            in_specs=[g_spec],
            out_specs=out_spec,
            out_shape=out_shape,
        )
    return pl.pallas_call(
        _tc_transpose_body,
        grid=(SSEQ,),
        in_specs=[pl.BlockSpec(memory_space=pl.ANY), g_spec],
        out_specs=out_spec,
        out_shape=out_shape,
        input_output_aliases={0: 0},
    )


_tc_transpose = [_make_tc_transpose(i * SSEQ, i == 0) for i in range(NSLICE)]


def kernel(x, tables, phases, amplitudes):
    xt = x.T.astype(jnp.int32)
    tbl = tables.transpose(1, 0, 2).reshape(VOCAB, C)
    coef = (jnp.cos(phases) * amplitudes[:, None]).astype(jnp.float32)
    coef = coef.reshape(C)

    g = [_sc_call[i](tbl, xt, coef) for i in range(NSLICE)]
    out = _tc_transpose[0](g[0].reshape(SSEQ, BATCH, C))
    for i in range(1, NSLICE):
        out = _tc_transpose[i](out, g[i].reshape(SSEQ, BATCH, C))
    out = out.reshape(SEQ, N_HYP, D, BATCH)
    return out.transpose(3, 0, 1, 2)


# speedup vs baseline: 4.9757x; 1.0618x over previous
import functools

import jax
import jax.numpy as jnp
from jax import lax
from jax.experimental import pallas as pl
from jax.experimental.pallas import tpu as pltpu
from jax.experimental.pallas import tpu_sc as plsc

VOCAB = 100000
D = 64
N_HYP = 4
BATCH = 1024
SEQ = 200

C = N_HYP * D
NW = 32
CHUNK = 128
NBBLK = BATCH // CHUNK
NSLICE = 5
SSEQ = SEQ // NSLICE
UPW = SSEQ * NBBLK // NW
PAIRS = UPW // 2


def _make_sc_kernel(s0):
    mesh = plsc.VectorSubcoreMesh(core_axis_name="c", subcore_axis_name="s")

    @functools.partial(
        pl.kernel,
        mesh=mesh,
        out_type=jax.ShapeDtypeStruct((SSEQ * BATCH, C), jnp.float32),
        scratch_types=[
            pltpu.VMEM((UPW * CHUNK,), jnp.int32),
            pltpu.VMEM((CHUNK, C), jnp.float32),
            pltpu.VMEM((CHUNK, C), jnp.float32),
            pltpu.SemaphoreType.DMA,
            pltpu.SemaphoreType.DMA,
            pltpu.SemaphoreType.DMA,
            pltpu.SemaphoreType.DMA,
        ],
    )
    def k(tbl_hbm, xtf_hbm, out_hbm, idx_all, rows0, rows1,
          sg0, sg1, so0, so1):
        nc = 2
        wid = lax.axis_index("s") * nc + lax.axis_index("c")
        tok0 = wid * (UPW * CHUNK)

        pltpu.sync_copy(
            xtf_hbm.at[pl.ds(s0 * BATCH + tok0, UPW * CHUNK)], idx_all)

        def idxs(u):
            return idx_all.at[pl.ds(u * CHUNK, CHUNK)]

        def dst(u):
            return out_hbm.at[pl.ds(tok0 + u * CHUNK, CHUNK)]

        pltpu.async_copy(tbl_hbm.at[idxs(0)], rows0, sg0)

        def body(i, carry):
            ua = 2 * i
            ub = ua + 1

            @pl.when(i > 0)
            def _():
                pltpu.make_async_copy(rows1, dst(ub - 2), so1).wait()

            pltpu.async_copy(tbl_hbm.at[idxs(ub)], rows1, sg1)
            pltpu.make_async_copy(tbl_hbm.at[idxs(ua)], rows0, sg0).wait()
            pltpu.async_copy(rows0, dst(ua), so0)

            @pl.when(i < PAIRS - 1)
            def _():
                pltpu.make_async_copy(rows0, dst(ua), so0).wait()
                pltpu.async_copy(tbl_hbm.at[idxs(ua + 2)], rows0, sg0)

            pltpu.make_async_copy(tbl_hbm.at[idxs(ub)], rows1, sg1).wait()
            pltpu.async_copy(rows1, dst(ub), so1)
            return carry

        lax.fori_loop(0, PAIRS, body, 0)
        pltpu.make_async_copy(rows0, dst(UPW - 2), so0).wait()
        pltpu.make_async_copy(rows1, dst(UPW - 1), so1).wait()

    return k


_sc_call = [_make_sc_kernel(i * SSEQ) for i in range(NSLICE)]


def _tc_transpose_body_first(g_ref, coef_ref, out_ref):
    out_ref[0] = g_ref[0].T * coef_ref[0][:, None]


def _tc_transpose_body(carry_ref, g_ref, coef_ref, out_ref):
    del carry_ref
    out_ref[0] = g_ref[0].T * coef_ref[0][:, None]


def _make_tc_transpose(s0, first):
    g_spec = pl.BlockSpec((1, BATCH, C), lambda j: (j, 0, 0))
    coef_spec = pl.BlockSpec((1, C), lambda j: (0, 0))
    out_spec = pl.BlockSpec((1, C, BATCH), lambda j: (s0 + j, 0, 0))
    out_shape = jax.ShapeDtypeStruct((SEQ, C, BATCH), jnp.float32)
    if first:
        return pl.pallas_call(
            _tc_transpose_body_first,
            grid=(SSEQ,),
            in_specs=[g_spec, coef_spec],
            out_specs=out_spec,
            out_shape=out_shape,
        )
    return pl.pallas_call(
        _tc_transpose_body,
        grid=(SSEQ,),
        in_specs=[pl.BlockSpec(memory_space=pl.ANY), g_spec, coef_spec],
        out_specs=out_spec,
        out_shape=out_shape,
        input_output_aliases={0: 0},
    )


_tc_transpose = [_make_tc_transpose(i * SSEQ, i == 0) for i in range(NSLICE)]


def kernel(x, tables, phases, amplitudes):
    xtf = x.T.astype(jnp.int32).reshape(SEQ * BATCH)
    tbl = tables.transpose(1, 0, 2).reshape(VOCAB, C)
    coef = (jnp.cos(phases) * amplitudes[:, None]).astype(jnp.float32)
    coef = coef.reshape(1, C)

    g = [_sc_call[i](tbl, xtf) for i in range(NSLICE)]
    out = _tc_transpose[0](g[0].reshape(SSEQ, BATCH, C), coef)
    for i in range(1, NSLICE):
        out = _tc_transpose[i](out, g[i].reshape(SSEQ, BATCH, C), coef)
    out = out.reshape(SEQ, N_HYP, D, BATCH)
    return out.transpose(3, 0, 1, 2)
